# baseline (speedup 1.0000x reference)
import functools

import jax
import jax.numpy as jnp
from jax import lax
from jax.experimental import pallas as pl
from jax.experimental.pallas import tpu as pltpu
import jax.experimental.pallas.tpu_sc as plsc

N = 10000
E = 320000
D = 128
DOUT = 40

NC = 2
NS = 16
NW = NC * NS
CH = 128
CHUNKS = 80
EPW = CH * CHUNKS
EPAD = EPW * NW
NBUF = 4
HALF = 5120
HALFP = 6144
DUMP = 5632
RPT = HALFP // NS

_f32 = jnp.float32
_i32 = jnp.int32


def _zero_vmem_2d(buf, rows, width):
    zero16 = jnp.zeros((16,), _f32)

    def body(i, _):
        r = i // (width // 16)
        c = (i % (width // 16)) * 16
        buf[r, pl.ds(c, 16)] = zero16
        return 0

    lax.fori_loop(0, rows * (width // 16), body, 0)


def _sc_scatter_body(z_hbm, row_hbm, col_hbm, acc_out, ridx_v,
                     cb0, cb1, cb2, cb3, gbufs, acc_sh, gsem, csem):
    cid = lax.axis_index("c")
    sid = lax.axis_index("s")
    wid = cid * NS + sid
    cbufs = [cb0, cb1, cb2, cb3]

    pltpu.sync_copy(row_hbm.at[wid], ridx_v)

    dump16 = jnp.full((16,), DUMP, _i32)
    zero16i = jnp.zeros((16,), _i32)
    half16 = jnp.full((16,), HALF, _i32)

    for half in range(2):
        _zero_vmem_2d(gbufs.at[0], CH, D)
        for k in range(RPT // CH):
            pltpu.sync_copy(gbufs.at[0],
                            acc_sh.at[pl.ds(sid * RPT + k * CH, CH)])
        plsc.subcore_barrier()

        base16 = jnp.full((16,), half * HALF, _i32)

        def remap(cb):
            for j in range(CH // 16):
                c16 = cb[pl.ds(j * 16, 16)]
                t = c16 - base16
                ok = (t >= zero16i) & (t < half16)
                cb[pl.ds(j * 16, 16)] = jnp.where(ok, t, dump16)

        for b in range(NBUF):
            pltpu.async_copy(col_hbm.at[wid, b], cbufs[b], csem.at[b])
            pltpu.async_copy(z_hbm.at[ridx_v.at[b]], gbufs.at[b], gsem.at[b])

        def step(i, b, refill):
            pltpu.make_async_copy(
                col_hbm.at[wid, i], cbufs[b], csem.at[b]).wait()
            pltpu.make_async_copy(
                z_hbm.at[ridx_v.at[i]], gbufs.at[b], gsem.at[b]).wait()
            remap(cbufs[b])
            pltpu.sync_copy(gbufs.at[b], acc_sh.at[cbufs[b]], add=True)
            if refill:
                nxt = i + NBUF
                pltpu.async_copy(col_hbm.at[wid, nxt], cbufs[b], csem.at[b])
                pltpu.async_copy(
                    z_hbm.at[ridx_v.at[nxt]], gbufs.at[b], gsem.at[b])

        def group(g, _):
            for b in range(NBUF):
                step(g * NBUF + b, b, True)
            return 0

        lax.fori_loop(0, CHUNKS // NBUF - 1, group, 0)
        for b in range(NBUF):
            step(CHUNKS - NBUF + b, b, False)
        plsc.subcore_barrier()

        pltpu.sync_copy(acc_sh.at[pl.ds(sid * RPT, RPT)],
                        acc_out.at[cid, half, pl.ds(sid * RPT, RPT)])
        plsc.subcore_barrier()


@functools.lru_cache(maxsize=None)
def _sc_scatter():
    mesh = plsc.VectorSubcoreMesh(
        core_axis_name="c", subcore_axis_name="s",
        num_cores=NC, num_subcores=NS)
    return pl.kernel(
        _sc_scatter_body,
        out_type=jax.ShapeDtypeStruct((NC, 2, HALFP, D), _f32),
        mesh=mesh,
        scratch_types=[
            pltpu.VMEM((CHUNKS, CH), _i32),
            pltpu.VMEM((CH,), _i32),
            pltpu.VMEM((CH,), _i32),
            pltpu.VMEM((CH,), _i32),
            pltpu.VMEM((CH,), _i32),
            pltpu.VMEM((NBUF, CH, D), _f32),
            pltpu.VMEM_SHARED((HALFP, D), _f32),
            pltpu.SemaphoreType.DMA((NBUF,)),
            pltpu.SemaphoreType.DMA((NBUF,)),
        ],
    )


def _tc_update_body(acc_ref, cnt_ref, h_ref, wu_ref, g_ref, be_ref,
                    wn_ref, bn_ref, ubn_ref, isc_ref,
                    h_out, z_out, cnt_out):
    top = acc_ref[0, 0, :HALF, :] + acc_ref[1, 0, :HALF, :]
    bot = acc_ref[0, 1, :N - HALF, :] + acc_ref[1, 1, :N - HALF, :]
    accs = jnp.concatenate([top, bot], axis=0)
    isc = isc_ref[...]
    ubn = ubn_ref[...]
    cnt = jnp.where(isc > 0.0, accs, cnt_ref[...])
    aggr = accs / jnp.maximum(cnt, 1.0)
    u = jnp.maximum(
        jnp.dot(aggr, wu_ref[:D, :], preferred_element_type=_f32)
        + jnp.dot(h_ref[...], wu_ref[D:, :], preferred_element_type=_f32),
        0.0)
    mu = jnp.mean(u, axis=0, keepdims=True)
    var = jnp.mean((u - mu) * (u - mu), axis=0, keepdims=True)
    hbn = jnp.maximum(
        g_ref[...] * (u - mu) / jnp.sqrt(var + 1e-5) + be_ref[...], 0.0)
    hb = jnp.where(ubn > 0.0, hbn, u)
    h_new = jnp.where(isc > 0.0, h_ref[...], hb)
    h_out[...] = h_new
    z_out[...] = jnp.maximum(
        jnp.dot(h_new, wn_ref[...], preferred_element_type=_f32)
        + bn_ref[...], 0.0)
    cnt_out[...] = cnt


def _tc_final_body(h_ref, wo_ref, bo_ref, o_ref):
    o_ref[...] = jnp.dot(h_ref[...], wo_ref[...],
                         preferred_element_type=_f32) + bo_ref[...]


_tc_update = pl.pallas_call(
    _tc_update_body,
    out_shape=[jax.ShapeDtypeStruct((N, D), _f32),
               jax.ShapeDtypeStruct((N, D), _f32),
               jax.ShapeDtypeStruct((N, D), _f32)])

_tc_final = pl.pallas_call(
    _tc_final_body, out_shape=jax.ShapeDtypeStruct((N, D), _f32))


def kernel(x, edge_index, pos,
           W_lin1, b_lin1, W_upd1,
           W_lin2, b_lin2, W_upd2,
           W_lin3, b_lin3, W_upd3,
           bn_gamma, bn_beta, W_out, b_out):
    row = edge_index[0].astype(_i32)
    col = edge_index[1].astype(_i32)
    npad = EPAD - E
    rowp = jnp.concatenate([row, jnp.zeros((npad,), _i32)])
    colp = jnp.concatenate([col, jnp.full((npad,), N, _i32)])
    rowp = rowp.reshape(NW, CHUNKS, CH)
    colp = colp.reshape(NW, CHUNKS, CH)

    g = bn_gamma.reshape(1, D)
    be = bn_beta.reshape(1, D)
    zero_w = jnp.zeros((D, D), _f32)
    zero_b = jnp.zeros((1, D), _f32)
    wo = jnp.zeros((D, D), _f32).at[:, :DOUT].set(W_out)
    bo = jnp.zeros((1, D), _f32).at[0, :DOUT].set(b_out)

    wu = jnp.stack([jnp.zeros((2 * D, D), _f32), W_upd1, W_upd2, W_upd3])
    wn = jnp.stack([W_lin1, W_lin2, W_lin3, zero_w])
    bn = jnp.stack([b_lin1.reshape(1, D), b_lin2.reshape(1, D),
                    b_lin3.reshape(1, D), zero_b])
    ubn = jnp.stack([zero_b, zero_b + 1.0, zero_b + 1.0, zero_b])
    isc = jnp.stack([zero_b + 1.0, zero_b, zero_b, zero_b])

    sc_scatter = _sc_scatter()

    def body(carry, xs):
        h, z, cnt = carry
        acc = sc_scatter(z, rowp, colp)
        h2, z2, cnt2 = _tc_update(acc, cnt, h, xs['wu'], g, be,
                                  xs['wn'], xs['bn'], xs['ubn'], xs['isc'])
        return (h2, z2, cnt2), 0.0

    init = (x, jnp.ones((N, D), _f32), jnp.ones((N, D), _f32))
    (h_fin, _, _), _ = lax.scan(
        body, init, {'wu': wu, 'wn': wn, 'bn': bn, 'ubn': ubn, 'isc': isc})

    out = _tc_final(h_fin, wo, bo)
    return out[:, :DOUT]


# speedup vs baseline: 2.7883x; 2.7883x over previous
import functools

import jax
import jax.numpy as jnp
from jax import lax
from jax.experimental import pallas as pl
from jax.experimental.pallas import tpu as pltpu
import jax.experimental.pallas.tpu_sc as plsc

N = 10000
E = 320000
D = 128
DOUT = 40

NC = 2
NS = 16
NW = NC * NS
CH = 128
CHUNKS = 80
EPW = CH * CHUNKS
EPAD = EPW * NW
NPAD = 10240
RPT = NPAD // NS

_f32 = jnp.float32
_i32 = jnp.int32


def _zero_vmem_2d(buf, rows, width):
    zero16 = jnp.zeros((16,), _f32)

    def body(i, _):
        r = i // (width // 16)
        c = (i % (width // 16)) * 16
        buf[r, pl.ds(c, 16)] = zero16
        return 0

    lax.fori_loop(0, rows * (width // 16), body, 0)


def _sc_scatter_body(z_hbm, row_hbm, col_hbm, acc_out, ridx0, ridx1, cidx_v,
                     gbuf0, gbuf1, acc_sh, gsem0, gsem1):
    cid = lax.axis_index("c")
    sid = lax.axis_index("s")
    wid = cid * NS + sid

    _zero_vmem_2d(gbuf0, CH, D)
    for k in range(RPT // CH):
        pltpu.sync_copy(gbuf0, acc_sh.at[pl.ds(sid * RPT + k * CH, CH)])
    plsc.subcore_barrier()

    base_w = wid * EPW

    def fire(base, ridx, gbuf, gsem):
        pltpu.sync_copy(row_hbm.at[pl.ds(base, CH)], ridx)
        pltpu.async_copy(z_hbm.at[ridx], gbuf, gsem)

    def drain(ridx, gbuf, gsem, base):
        pltpu.make_async_copy(z_hbm.at[ridx], gbuf, gsem).wait()
        pltpu.sync_copy(col_hbm.at[pl.ds(base, CH)], cidx_v)
        pltpu.sync_copy(gbuf, acc_sh.at[cidx_v], add=True)

    fire(base_w, ridx0, gbuf0, gsem0)

    def pair(g, _):
        base = base_w + 2 * g * CH
        fire(base + CH, ridx1, gbuf1, gsem1)
        drain(ridx0, gbuf0, gsem0, base)
        fire(base + 2 * CH, ridx0, gbuf0, gsem0)
        drain(ridx1, gbuf1, gsem1, base + CH)
        return 0

    lax.fori_loop(0, CHUNKS // 2 - 1, pair, 0)
    base = base_w + (CHUNKS - 2) * CH
    fire(base + CH, ridx1, gbuf1, gsem1)
    drain(ridx0, gbuf0, gsem0, base)
    drain(ridx1, gbuf1, gsem1, base + CH)
    plsc.subcore_barrier()

    pltpu.sync_copy(acc_sh.at[pl.ds(sid * RPT, RPT)],
                    acc_out.at[cid, pl.ds(sid * RPT, RPT)])


def _sc_count_body(col_hbm, cnt_out, cidx_v, ones_v, zbuf, cnt_sh):
    cid = lax.axis_index("c")
    sid = lax.axis_index("s")
    wid = cid * NS + sid

    _zero_vmem_2d(zbuf, CH, D)
    one16 = jnp.ones((16,), _f32)

    def fill_ones(i, _):
        r = i // (D // 16)
        c = (i % (D // 16)) * 16
        ones_v[r, pl.ds(c, 16)] = one16
        return 0

    lax.fori_loop(0, CH * (D // 16), fill_ones, 0)

    for k in range(RPT // CH):
        pltpu.sync_copy(zbuf, cnt_sh.at[pl.ds(sid * RPT + k * CH, CH)])
    plsc.subcore_barrier()

    base_w = wid * EPW

    def chunk(i, _):
        base = base_w + i * CH
        pltpu.sync_copy(col_hbm.at[pl.ds(base, CH)], cidx_v)
        pltpu.sync_copy(ones_v, cnt_sh.at[cidx_v], add=True)
        return 0

    lax.fori_loop(0, CHUNKS, chunk, 0)
    plsc.subcore_barrier()

    pltpu.sync_copy(cnt_sh.at[pl.ds(sid * RPT, RPT)],
                    cnt_out.at[cid, pl.ds(sid * RPT, RPT)])


@functools.lru_cache(maxsize=None)
def _sc_kernels():
    mesh = plsc.VectorSubcoreMesh(
        core_axis_name="c", subcore_axis_name="s",
        num_cores=NC, num_subcores=NS)

    scatter = pl.kernel(
        _sc_scatter_body,
        out_type=jax.ShapeDtypeStruct((NC, NPAD, D), _f32),
        mesh=mesh,
        scratch_types=[
            pltpu.VMEM((CH,), _i32),
            pltpu.VMEM((CH,), _i32),
            pltpu.VMEM((CH,), _i32),
            pltpu.VMEM((CH, D), _f32),
            pltpu.VMEM((CH, D), _f32),
            pltpu.VMEM_SHARED((NPAD, D), _f32),
            pltpu.SemaphoreType.DMA,
            pltpu.SemaphoreType.DMA,
        ],
    )

    count = pl.kernel(
        _sc_count_body,
        out_type=jax.ShapeDtypeStruct((NC, NPAD, D), _f32),
        mesh=mesh,
        scratch_types=[
            pltpu.VMEM((CH,), _i32),
            pltpu.VMEM((CH, D), _f32),
            pltpu.VMEM((CH, D), _f32),
            pltpu.VMEM_SHARED((NPAD, D), _f32),
        ],
    )
    return scatter, count


def _tc_lin_body(x_ref, w_ref, b_ref, z_ref):
    z_ref[...] = jnp.maximum(
        jnp.dot(x_ref[...], w_ref[...], preferred_element_type=_f32)
        + b_ref[...], 0.0)


def _tc_update_body(acc_ref, cnt_ref, x_ref, wu_ref, g_ref, be_ref,
                    wn_ref, bn_ref, h_ref, z_ref):
    acc = acc_ref[0, :N, :] + acc_ref[1, :N, :]
    cnt = cnt_ref[0, :N, :1] + cnt_ref[1, :N, :1]
    aggr = acc / jnp.maximum(cnt, 1.0)
    u = jnp.maximum(
        jnp.dot(aggr, wu_ref[:D, :], preferred_element_type=_f32)
        + jnp.dot(x_ref[...], wu_ref[D:, :], preferred_element_type=_f32),
        0.0)
    mu = jnp.mean(u, axis=0, keepdims=True)
    var = jnp.mean((u - mu) * (u - mu), axis=0, keepdims=True)
    h = jnp.maximum(
        g_ref[...] * (u - mu) / jnp.sqrt(var + 1e-5) + be_ref[...], 0.0)
    h_ref[...] = h
    z_ref[...] = jnp.maximum(
        jnp.dot(h, wn_ref[...], preferred_element_type=_f32) + bn_ref[...],
        0.0)


def _tc_final_body(acc_ref, cnt_ref, x_ref, wu_ref, wo_ref, bo_ref, o_ref):
    acc = acc_ref[0, :N, :] + acc_ref[1, :N, :]
    cnt = cnt_ref[0, :N, :1] + cnt_ref[1, :N, :1]
    aggr = acc / jnp.maximum(cnt, 1.0)
    u = jnp.maximum(
        jnp.dot(aggr, wu_ref[:D, :], preferred_element_type=_f32)
        + jnp.dot(x_ref[...], wu_ref[D:, :], preferred_element_type=_f32),
        0.0)
    o_ref[...] = jnp.dot(u, wo_ref[...], preferred_element_type=_f32) \
        + bo_ref[...]


_tc_lin = pl.pallas_call(
    _tc_lin_body, out_shape=jax.ShapeDtypeStruct((N, D), _f32))

_tc_update = pl.pallas_call(
    _tc_update_body,
    out_shape=[jax.ShapeDtypeStruct((N, D), _f32),
               jax.ShapeDtypeStruct((N, D), _f32)])

_tc_final = pl.pallas_call(
    _tc_final_body, out_shape=jax.ShapeDtypeStruct((N, D), _f32))


def kernel(x, edge_index, pos,
           W_lin1, b_lin1, W_upd1,
           W_lin2, b_lin2, W_upd2,
           W_lin3, b_lin3, W_upd3,
           bn_gamma, bn_beta, W_out, b_out):
    row = edge_index[0].astype(_i32)
    col = edge_index[1].astype(_i32)
    npad = EPAD - E
    rowp = jnp.concatenate([row, jnp.zeros((npad,), _i32)])
    colp = jnp.concatenate([col, jnp.full((npad,), N, _i32)])

    b1 = b_lin1.reshape(1, D)
    b2 = b_lin2.reshape(1, D)
    b3 = b_lin3.reshape(1, D)
    g = bn_gamma.reshape(1, D)
    be = bn_beta.reshape(1, D)
    wo = jnp.zeros((D, D), _f32).at[:, :DOUT].set(W_out)
    bo = jnp.zeros((1, D), _f32).at[0, :DOUT].set(b_out)

    _sc_scatter, _sc_count = _sc_kernels()
    cnt = _sc_count(colp)

    z1 = _tc_lin(x, W_lin1, b1)
    acc1 = _sc_scatter(z1, rowp, colp)
    h1, z2 = _tc_update(acc1, cnt, x, W_upd1, g, be, W_lin2, b2)
    acc2 = _sc_scatter(z2, rowp, colp)
    h2, z3 = _tc_update(acc2, cnt, h1, W_upd2, g, be, W_lin3, b3)
    acc3 = _sc_scatter(z3, rowp, colp)
    out = _tc_final(acc3, cnt, h2, W_upd3, wo, bo)
    return out[:, :DOUT]


# speedup vs baseline: 2.8007x; 1.0045x over previous
import functools

import jax
import jax.numpy as jnp
from jax import lax
from jax.experimental import pallas as pl
from jax.experimental.pallas import tpu as pltpu
import jax.experimental.pallas.tpu_sc as plsc

N = 10000
E = 320000
D = 128
DOUT = 40

NC = 2
NS = 16
NW = NC * NS
CH = 128
CHUNKS = 80
EPW = CH * CHUNKS
EPAD = EPW * NW
NPAD = 10112
RPT = NPAD // NS

_f32 = jnp.float32
_i32 = jnp.int32


def _zero_vmem_2d(buf, rows, width):
    zero16 = jnp.zeros((16,), _f32)

    def body(i, _):
        r = i // (width // 16)
        c = (i % (width // 16)) * 16
        buf[r, pl.ds(c, 16)] = zero16
        return 0

    lax.fori_loop(0, rows * (width // 16), body, 0)


def _sc_scatter_body(z_hbm, row_hbm, col_hbm, acc_out,
                     ridx0, ridx1, ridx2, cidx_v,
                     gbuf0, gbuf1, gbuf2, acc_sh,
                     gsem0, gsem1, gsem2):
    cid = lax.axis_index("c")
    sid = lax.axis_index("s")
    wid = cid * NS + sid
    bufs = [(ridx0, gbuf0, gsem0), (ridx1, gbuf1, gsem1),
            (ridx2, gbuf2, gsem2)]

    _zero_vmem_2d(gbuf0, CH, D)
    for k in range(RPT // CH):
        pltpu.sync_copy(gbuf0, acc_sh.at[pl.ds(sid * RPT + k * CH, CH)])
    pltpu.sync_copy(gbuf0.at[pl.ds(0, RPT % CH)],
                    acc_sh.at[pl.ds(sid * RPT + (RPT // CH) * CH, RPT % CH)])
    plsc.subcore_barrier()

    base_w = wid * EPW

    def fire(i, t):
        ridx, gbuf, gsem = t
        pltpu.sync_copy(row_hbm.at[pl.ds(base_w + i * CH, CH)], ridx)
        pltpu.async_copy(z_hbm.at[ridx], gbuf, gsem)

    def drain(i, t):
        ridx, gbuf, gsem = t
        pltpu.make_async_copy(z_hbm.at[ridx], gbuf, gsem).wait()
        pltpu.sync_copy(col_hbm.at[pl.ds(base_w + i * CH, CH)], cidx_v)
        pltpu.sync_copy(gbuf, acc_sh.at[cidx_v], add=True)

    fire(0, bufs[0])
    fire(1, bufs[1])

    def tri(g, _):
        for k in range(3):
            i = 3 * g + k
            fire(i + 2, bufs[(k + 2) % 3])
            drain(i, bufs[k])
        return 0

    lax.fori_loop(0, CHUNKS // 3 - 1, tri, 0)
    base = (CHUNKS // 3 - 1) * 3
    for i in range(base, CHUNKS):
        if i + 2 < CHUNKS:
            fire(i + 2, bufs[(i + 2) % 3])
        drain(i, bufs[i % 3])
    plsc.subcore_barrier()

    pltpu.sync_copy(acc_sh.at[pl.ds(sid * RPT, RPT)],
                    acc_out.at[cid, pl.ds(sid * RPT, RPT)])


def _sc_count_body(col_hbm, cnt_out, cidx_v, ones_v, zbuf, cnt_sh):
    cid = lax.axis_index("c")
    sid = lax.axis_index("s")
    wid = cid * NS + sid

    _zero_vmem_2d(zbuf, CH, D)
    one16 = jnp.ones((16,), _f32)

    def fill_ones(i, _):
        r = i // (D // 16)
        c = (i % (D // 16)) * 16
        ones_v[r, pl.ds(c, 16)] = one16
        return 0

    lax.fori_loop(0, CH * (D // 16), fill_ones, 0)

    for k in range(RPT // CH):
        pltpu.sync_copy(zbuf, cnt_sh.at[pl.ds(sid * RPT + k * CH, CH)])
    pltpu.sync_copy(zbuf.at[pl.ds(0, RPT % CH)],
                    cnt_sh.at[pl.ds(sid * RPT + (RPT // CH) * CH, RPT % CH)])
    plsc.subcore_barrier()

    base_w = wid * EPW

    def chunk(i, _):
        base = base_w + i * CH
        pltpu.sync_copy(col_hbm.at[pl.ds(base, CH)], cidx_v)
        pltpu.sync_copy(ones_v, cnt_sh.at[cidx_v], add=True)
        return 0

    lax.fori_loop(0, CHUNKS, chunk, 0)
    plsc.subcore_barrier()

    pltpu.sync_copy(cnt_sh.at[pl.ds(sid * RPT, RPT)],
                    cnt_out.at[cid, pl.ds(sid * RPT, RPT)])


@functools.lru_cache(maxsize=None)
def _sc_kernels():
    mesh = plsc.VectorSubcoreMesh(
        core_axis_name="c", subcore_axis_name="s",
        num_cores=NC, num_subcores=NS)

    scatter = pl.kernel(
        _sc_scatter_body,
        out_type=jax.ShapeDtypeStruct((NC, NPAD, D), _f32),
        mesh=mesh,
        scratch_types=[
            pltpu.VMEM((CH,), _i32),
            pltpu.VMEM((CH,), _i32),
            pltpu.VMEM((CH,), _i32),
            pltpu.VMEM((CH,), _i32),
            pltpu.VMEM((CH, D), _f32),
            pltpu.VMEM((CH, D), _f32),
            pltpu.VMEM((CH, D), _f32),
            pltpu.VMEM_SHARED((NPAD, D), _f32),
            pltpu.SemaphoreType.DMA,
            pltpu.SemaphoreType.DMA,
            pltpu.SemaphoreType.DMA,
        ],
    )

    count = pl.kernel(
        _sc_count_body,
        out_type=jax.ShapeDtypeStruct((NC, NPAD, D), _f32),
        mesh=mesh,
        scratch_types=[
            pltpu.VMEM((CH,), _i32),
            pltpu.VMEM((CH, D), _f32),
            pltpu.VMEM((CH, D), _f32),
            pltpu.VMEM_SHARED((NPAD, D), _f32),
        ],
    )
    return scatter, count


def _tc_lin_body(x_ref, w_ref, b_ref, z_ref):
    z_ref[...] = jnp.maximum(
        jnp.dot(x_ref[...], w_ref[...], preferred_element_type=_f32)
        + b_ref[...], 0.0)


def _tc_update_body(acc_ref, cnt_ref, x_ref, wu_ref, g_ref, be_ref,
                    wn_ref, bn_ref, h_ref, z_ref):
    acc = acc_ref[0, :N, :] + acc_ref[1, :N, :]
    cnt = cnt_ref[0, :N, :1] + cnt_ref[1, :N, :1]
    aggr = acc / jnp.maximum(cnt, 1.0)
    u = jnp.maximum(
        jnp.dot(aggr, wu_ref[:D, :], preferred_element_type=_f32)
        + jnp.dot(x_ref[...], wu_ref[D:, :], preferred_element_type=_f32),
        0.0)
    mu = jnp.mean(u, axis=0, keepdims=True)
    var = jnp.mean((u - mu) * (u - mu), axis=0, keepdims=True)
    h = jnp.maximum(
        g_ref[...] * (u - mu) / jnp.sqrt(var + 1e-5) + be_ref[...], 0.0)
    h_ref[...] = h
    z_ref[...] = jnp.maximum(
        jnp.dot(h, wn_ref[...], preferred_element_type=_f32) + bn_ref[...],
        0.0)


def _tc_final_body(acc_ref, cnt_ref, x_ref, wu_ref, wo_ref, bo_ref, o_ref):
    acc = acc_ref[0, :N, :] + acc_ref[1, :N, :]
    cnt = cnt_ref[0, :N, :1] + cnt_ref[1, :N, :1]
    aggr = acc / jnp.maximum(cnt, 1.0)
    u = jnp.maximum(
        jnp.dot(aggr, wu_ref[:D, :], preferred_element_type=_f32)
        + jnp.dot(x_ref[...], wu_ref[D:, :], preferred_element_type=_f32),
        0.0)
    o_ref[...] = jnp.dot(u, wo_ref[...], preferred_element_type=_f32) \
        + bo_ref[...]


_tc_lin = pl.pallas_call(
    _tc_lin_body, out_shape=jax.ShapeDtypeStruct((N, D), _f32))

_tc_update = pl.pallas_call(
    _tc_update_body,
    out_shape=[jax.ShapeDtypeStruct((N, D), _f32),
               jax.ShapeDtypeStruct((N, D), _f32)])

_tc_final = pl.pallas_call(
    _tc_final_body, out_shape=jax.ShapeDtypeStruct((N, D), _f32))


def kernel(x, edge_index, pos,
           W_lin1, b_lin1, W_upd1,
           W_lin2, b_lin2, W_upd2,
           W_lin3, b_lin3, W_upd3,
           bn_gamma, bn_beta, W_out, b_out):
    row = edge_index[0].astype(_i32)
    col = edge_index[1].astype(_i32)
    npad = EPAD - E
    rowp = jnp.concatenate([row, jnp.zeros((npad,), _i32)])
    colp = jnp.concatenate([col, jnp.full((npad,), N, _i32)])

    b1 = b_lin1.reshape(1, D)
    b2 = b_lin2.reshape(1, D)
    b3 = b_lin3.reshape(1, D)
    g = bn_gamma.reshape(1, D)
    be = bn_beta.reshape(1, D)
    wo = jnp.zeros((D, D), _f32).at[:, :DOUT].set(W_out)
    bo = jnp.zeros((1, D), _f32).at[0, :DOUT].set(b_out)

    _sc_scatter, _sc_count = _sc_kernels()
    cnt = _sc_count(colp)

    z1 = _tc_lin(x, W_lin1, b1)
    acc1 = _sc_scatter(z1, rowp, colp)
    h1, z2 = _tc_update(acc1, cnt, x, W_upd1, g, be, W_lin2, b2)
    acc2 = _sc_scatter(z2, rowp, colp)
    h2, z3 = _tc_update(acc2, cnt, h1, W_upd2, g, be, W_lin3, b3)
    acc3 = _sc_scatter(z3, rowp, colp)
    out = _tc_final(acc3, cnt, h2, W_upd3, wo, bo)
    return out[:, :DOUT]


# speedup vs baseline: 2.8606x; 1.0214x over previous
import functools

import jax
import jax.numpy as jnp
from jax import lax
from jax.experimental import pallas as pl
from jax.experimental.pallas import tpu as pltpu
import jax.experimental.pallas.tpu_sc as plsc

N = 10000
E = 320000
D = 128
DOUT = 40

NC = 2
NS = 16
NW = NC * NS
CH = 128
CHUNKS = 80
EPW = CH * CHUNKS
EPAD = EPW * NW
NPAD = 10112
RPT = NPAD // NS

_f32 = jnp.float32
_i32 = jnp.int32


def _zero_vmem_2d(buf, rows, width):
    zero16 = jnp.zeros((16,), _f32)

    def body(i, _):
        r = i // (width // 16)
        c = (i % (width // 16)) * 16
        buf[r, pl.ds(c, 16)] = zero16
        return 0

    lax.fori_loop(0, rows * (width // 16), body, 0)


def _sc_scatter_body(z_hbm, row_hbm, col_hbm, acc_out,
                     ridx0, ridx1, ridx2, cidx0, cidx1,
                     gbuf0, gbuf1, gbuf2, acc_sh,
                     gsem0, gsem1, gsem2, csem0, csem1):
    cid = lax.axis_index("c")
    sid = lax.axis_index("s")
    wid = cid * NS + sid
    bufs = [(ridx0, gbuf0, gsem0), (ridx1, gbuf1, gsem1),
            (ridx2, gbuf2, gsem2)]
    cps = [(cidx0, csem0), (cidx1, csem1)]

    _zero_vmem_2d(gbuf0, CH, D)
    for k in range(RPT // CH):
        pltpu.sync_copy(gbuf0, acc_sh.at[pl.ds(sid * RPT + k * CH, CH)])
    pltpu.sync_copy(gbuf0.at[pl.ds(0, RPT % CH)],
                    acc_sh.at[pl.ds(sid * RPT + (RPT // CH) * CH, RPT % CH)])
    plsc.subcore_barrier()

    base_w = wid * EPW

    def fire(i, t):
        ridx, gbuf, gsem = t
        pltpu.sync_copy(row_hbm.at[pl.ds(base_w + i * CH, CH)], ridx)
        pltpu.async_copy(z_hbm.at[ridx], gbuf, gsem)

    def fire_col(i, c):
        cidx, csem = c
        pltpu.async_copy(col_hbm.at[pl.ds(base_w + i * CH, CH)], cidx, csem)

    def drain(i, t, c, refill_col):
        ridx, gbuf, gsem = t
        cidx, csem = c
        pltpu.make_async_copy(z_hbm.at[ridx], gbuf, gsem).wait()
        pltpu.make_async_copy(
            col_hbm.at[pl.ds(base_w + i * CH, CH)], cidx, csem).wait()
        pltpu.sync_copy(gbuf, acc_sh.at[cidx], add=True)
        if refill_col:
            fire_col(i + 2, c)

    fire(0, bufs[0])
    fire(1, bufs[1])
    fire_col(0, cps[0])
    fire_col(1, cps[1])

    def sextet(g, _):
        for k in range(6):
            i = 6 * g + k
            fire(i + 2, bufs[(k + 2) % 3])
            drain(i, bufs[k % 3], cps[k % 2], True)
        return 0

    lax.fori_loop(0, CHUNKS // 6, sextet, 0)
    for i in range((CHUNKS // 6) * 6, CHUNKS):
        if i + 2 < CHUNKS:
            fire(i + 2, bufs[(i + 2) % 3])
        drain(i, bufs[i % 3], cps[i % 2], i + 2 < CHUNKS)
    plsc.subcore_barrier()

    pltpu.sync_copy(acc_sh.at[pl.ds(sid * RPT, RPT)],
                    acc_out.at[cid, pl.ds(sid * RPT, RPT)])


def _sc_count_body(col_hbm, cnt_out, cidx_v, ones_v, zbuf, cnt_sh):
    cid = lax.axis_index("c")
    sid = lax.axis_index("s")
    wid = cid * NS + sid

    _zero_vmem_2d(zbuf, CH, D)
    one16 = jnp.ones((16,), _f32)

    def fill_ones(i, _):
        r = i // (D // 16)
        c = (i % (D // 16)) * 16
        ones_v[r, pl.ds(c, 16)] = one16
        return 0

    lax.fori_loop(0, CH * (D // 16), fill_ones, 0)

    for k in range(RPT // CH):
        pltpu.sync_copy(zbuf, cnt_sh.at[pl.ds(sid * RPT + k * CH, CH)])
    pltpu.sync_copy(zbuf.at[pl.ds(0, RPT % CH)],
                    cnt_sh.at[pl.ds(sid * RPT + (RPT // CH) * CH, RPT % CH)])
    plsc.subcore_barrier()

    base_w = wid * EPW

    def chunk(i, _):
        base = base_w + i * CH
        pltpu.sync_copy(col_hbm.at[pl.ds(base, CH)], cidx_v)
        pltpu.sync_copy(ones_v, cnt_sh.at[cidx_v], add=True)
        return 0

    lax.fori_loop(0, CHUNKS, chunk, 0)
    plsc.subcore_barrier()

    pltpu.sync_copy(cnt_sh.at[pl.ds(sid * RPT, RPT)],
                    cnt_out.at[cid, pl.ds(sid * RPT, RPT)])


@functools.lru_cache(maxsize=None)
def _sc_kernels():
    mesh = plsc.VectorSubcoreMesh(
        core_axis_name="c", subcore_axis_name="s",
        num_cores=NC, num_subcores=NS)

    scatter = pl.kernel(
        _sc_scatter_body,
        out_type=jax.ShapeDtypeStruct((NC, NPAD, D), _f32),
        mesh=mesh,
        scratch_types=[
            pltpu.VMEM((CH,), _i32),
            pltpu.VMEM((CH,), _i32),
            pltpu.VMEM((CH,), _i32),
            pltpu.VMEM((CH,), _i32),
            pltpu.VMEM((CH,), _i32),
            pltpu.VMEM((CH, D), _f32),
            pltpu.VMEM((CH, D), _f32),
            pltpu.VMEM((CH, D), _f32),
            pltpu.VMEM_SHARED((NPAD, D), _f32),
            pltpu.SemaphoreType.DMA,
            pltpu.SemaphoreType.DMA,
            pltpu.SemaphoreType.DMA,
            pltpu.SemaphoreType.DMA,
            pltpu.SemaphoreType.DMA,
        ],
    )

    count = pl.kernel(
        _sc_count_body,
        out_type=jax.ShapeDtypeStruct((NC, NPAD, D), _f32),
        mesh=mesh,
        scratch_types=[
            pltpu.VMEM((CH,), _i32),
            pltpu.VMEM((CH, D), _f32),
            pltpu.VMEM((CH, D), _f32),
            pltpu.VMEM_SHARED((NPAD, D), _f32),
        ],
    )
    return scatter, count


def _tc_lin_body(x_ref, w_ref, b_ref, z_ref):
    z_ref[...] = jnp.maximum(
        jnp.dot(x_ref[...], w_ref[...], preferred_element_type=_f32)
        + b_ref[...], 0.0)


def _tc_update_body(acc_ref, cnt_ref, x_ref, wu_ref, g_ref, be_ref,
                    wn_ref, bn_ref, h_ref, z_ref):
    acc = acc_ref[0, :N, :] + acc_ref[1, :N, :]
    cnt = cnt_ref[0, :N, :1] + cnt_ref[1, :N, :1]
    aggr = acc / jnp.maximum(cnt, 1.0)
    u = jnp.maximum(
        jnp.dot(aggr, wu_ref[:D, :], preferred_element_type=_f32)
        + jnp.dot(x_ref[...], wu_ref[D:, :], preferred_element_type=_f32),
        0.0)
    mu = jnp.mean(u, axis=0, keepdims=True)
    var = jnp.mean((u - mu) * (u - mu), axis=0, keepdims=True)
    h = jnp.maximum(
        g_ref[...] * (u - mu) / jnp.sqrt(var + 1e-5) + be_ref[...], 0.0)
    h_ref[...] = h
    z_ref[...] = jnp.maximum(
        jnp.dot(h, wn_ref[...], preferred_element_type=_f32) + bn_ref[...],
        0.0)


def _tc_final_body(acc_ref, cnt_ref, x_ref, wu_ref, wo_ref, bo_ref, o_ref):
    acc = acc_ref[0, :N, :] + acc_ref[1, :N, :]
    cnt = cnt_ref[0, :N, :1] + cnt_ref[1, :N, :1]
    aggr = acc / jnp.maximum(cnt, 1.0)
    u = jnp.maximum(
        jnp.dot(aggr, wu_ref[:D, :], preferred_element_type=_f32)
        + jnp.dot(x_ref[...], wu_ref[D:, :], preferred_element_type=_f32),
        0.0)
    o_ref[...] = jnp.dot(u, wo_ref[...], preferred_element_type=_f32) \
        + bo_ref[...]


_tc_lin = pl.pallas_call(
    _tc_lin_body, out_shape=jax.ShapeDtypeStruct((N, D), _f32))

_tc_update = pl.pallas_call(
    _tc_update_body,
    out_shape=[jax.ShapeDtypeStruct((N, D), _f32),
               jax.ShapeDtypeStruct((N, D), _f32)])

_tc_final = pl.pallas_call(
    _tc_final_body, out_shape=jax.ShapeDtypeStruct((N, D), _f32))


def kernel(x, edge_index, pos,
           W_lin1, b_lin1, W_upd1,
           W_lin2, b_lin2, W_upd2,
           W_lin3, b_lin3, W_upd3,
           bn_gamma, bn_beta, W_out, b_out):
    row = edge_index[0].astype(_i32)
    col = edge_index[1].astype(_i32)
    npad = EPAD - E
    rowp = jnp.concatenate([row, jnp.zeros((npad,), _i32)])
    colp = jnp.concatenate([col, jnp.full((npad,), N, _i32)])

    b1 = b_lin1.reshape(1, D)
    b2 = b_lin2.reshape(1, D)
    b3 = b_lin3.reshape(1, D)
    g = bn_gamma.reshape(1, D)
    be = bn_beta.reshape(1, D)
    wo = jnp.zeros((D, D), _f32).at[:, :DOUT].set(W_out)
    bo = jnp.zeros((1, D), _f32).at[0, :DOUT].set(b_out)

    _sc_scatter, _sc_count = _sc_kernels()
    cnt = _sc_count(colp)

    z1 = _tc_lin(x, W_lin1, b1)
    acc1 = _sc_scatter(z1, rowp, colp)
    h1, z2 = _tc_update(acc1, cnt, x, W_upd1, g, be, W_lin2, b2)
    acc2 = _sc_scatter(z2, rowp, colp)
    h2, z3 = _tc_update(acc2, cnt, h1, W_upd2, g, be, W_lin3, b3)
    acc3 = _sc_scatter(z3, rowp, colp)
    out = _tc_final(acc3, cnt, h2, W_upd3, wo, bo)
    return out[:, :DOUT]
